# Initial kernel scaffold; baseline (speedup 1.0000x reference)
import jax
import jax.numpy as jnp
from jax.experimental import pallas as pl


def kernel(hidden_states, gate_w, W1, W2):
    raise NotImplementedError("write your pallas kernel here")


# speedup vs baseline: 1.3907x; 1.3907x over previous
import functools

import jax
import jax.numpy as jnp
from jax.experimental import pallas as pl

_T = 2048
_D = 768
_E = 8
_DFF = 3072
_EPAD = 128
_DFF_TILE = 512
_T_TILE = 512


def _routing_kernel(x_ref, gw_ref, c_ref, mrep_ref):
    x = x_ref[...]
    gw = gw_ref[...]
    g = jax.lax.dot_general(x, gw, (((1,), (1,)), ((), ())),
                            preferred_element_type=jnp.float32)
    lane = jax.lax.broadcasted_iota(jnp.int32, (_T, _EPAD), 1)
    neg = jnp.float32(-1e30)
    g = jnp.where(lane < _E, g, neg)
    m1 = jnp.max(g, axis=1, keepdims=True)
    a1 = jnp.min(jnp.where(g == m1, lane, _EPAD), axis=1, keepdims=True)
    g2 = jnp.where(lane == a1, neg, g)
    m2 = jnp.max(g2, axis=1, keepdims=True)
    a2 = jnp.min(jnp.where(g2 == m2, lane, _EPAD), axis=1, keepdims=True)
    w1 = jax.nn.sigmoid(m1 - m2)
    w2 = 1.0 - w1
    oh1 = (lane == a1)
    oh2 = (lane == a2)
    cnt = jnp.sum(oh1.astype(jnp.float32) + oh2.astype(jnp.float32),
                  axis=0, keepdims=True)
    inv = 1.0 / jnp.maximum(cnt, 1.0)
    c = (w1 * oh1.astype(jnp.float32) + w2 * oh2.astype(jnp.float32)) * inv
    c_ref[...] = c
    le = jax.lax.broadcasted_iota(jnp.int32, (_T, _E * _EPAD), 1) // _EPAD
    mrep_ref[...] = ((le == a1) | (le == a2)).astype(jnp.float32)


def _main_kernel(x_ref, w1_ref, mrep_ref, s1_ref):
    h = jax.lax.dot_general(x_ref[...], w1_ref[0], (((1,), (1,)), ((), ())),
                            preferred_element_type=jnp.float32)
    h = h * jax.nn.sigmoid(h)
    masked = h * mrep_ref[:, 0:1]
    s1_ref[0, 0, :] = jnp.sum(masked, axis=0)


def _mean_kernel(s1_ref, w2_ref, m_ref):
    s1 = s1_ref[0].astype(jnp.bfloat16)
    m_ref[0] = jax.lax.dot_general(
        s1, w2_ref[0], (((1,), (1,)), ((), ())),
        preferred_element_type=jnp.float32)


def _combine_kernel(c_ref, m_ref, out_ref):
    cs = c_ref[:, 0:_E]
    out_ref[...] = jax.lax.dot_general(
        cs, m_ref[...], (((1,), (0,)), ((), ())),
        preferred_element_type=jnp.float32)


@jax.jit
def kernel(hidden_states, gate_w, W1, W2):
    b, s, d = hidden_states.shape
    x = hidden_states.reshape(-1, d)

    gw_pad = jnp.zeros((_EPAD, _D), jnp.float32).at[:_E].set(gate_w)
    c, mrep = pl.pallas_call(
        _routing_kernel,
        out_shape=(
            jax.ShapeDtypeStruct((_T, _EPAD), jnp.float32),
            jax.ShapeDtypeStruct((_T, _E * _EPAD), jnp.float32),
        ),
    )(x, gw_pad)

    x16 = x.astype(jnp.bfloat16)
    w1_16 = W1.astype(jnp.bfloat16)
    s1 = pl.pallas_call(
        _main_kernel,
        grid=(_E, _DFF // _DFF_TILE),
        in_specs=[
            pl.BlockSpec((_T, _D), lambda e, j: (0, 0)),
            pl.BlockSpec((1, _DFF_TILE, _D), lambda e, j: (e, j, 0)),
            pl.BlockSpec((_T, _EPAD), lambda e, j: (0, e)),
        ],
        out_specs=pl.BlockSpec((1, 1, _DFF_TILE), lambda e, j: (e, 0, j)),
        out_shape=jax.ShapeDtypeStruct((_E, 1, _DFF), jnp.float32),
    )(x16, w1_16, mrep)

    w2_16 = W2.astype(jnp.bfloat16)
    m = pl.pallas_call(
        _mean_kernel,
        grid=(_E,),
        in_specs=[
            pl.BlockSpec((1, 1, _DFF), lambda e: (e, 0, 0)),
            pl.BlockSpec((1, _D, _DFF), lambda e: (e, 0, 0)),
        ],
        out_specs=pl.BlockSpec((1, 1, _D), lambda e: (e, 0, 0)),
        out_shape=jax.ShapeDtypeStruct((_E, 1, _D), jnp.float32),
    )(s1, w2_16)
    m = m.reshape(_E, _D)

    out = pl.pallas_call(
        _combine_kernel,
        grid=(_T // _T_TILE,),
        in_specs=[
            pl.BlockSpec((_T_TILE, _EPAD), lambda t: (t, 0)),
            pl.BlockSpec((_E, _D), lambda t: (0, 0)),
        ],
        out_specs=pl.BlockSpec((_T_TILE, _D), lambda t: (t, 0)),
        out_shape=jax.ShapeDtypeStruct((_T, _D), jnp.float32),
    )(c, m)

    return out.reshape(b, s, d)


# speedup vs baseline: 1.9369x; 1.3928x over previous
import functools

import jax
import jax.numpy as jnp
from jax.experimental import pallas as pl

_T = 2048
_D = 768
_E = 8
_DFF = 3072
_EPAD = 128
_DFF_TILE = 512
_T_TILE = 512


def _routing_kernel(x_ref, gw_ref, c_ref, mrep_ref, x16_ref):
    x = x_ref[...]
    x16_ref[...] = x.astype(jnp.bfloat16)
    gw = gw_ref[...]
    g = jax.lax.dot_general(x, gw, (((1,), (1,)), ((), ())),
                            preferred_element_type=jnp.float32)
    lane = jax.lax.broadcasted_iota(jnp.int32, (_T, _EPAD), 1)
    neg = jnp.float32(-1e30)
    g = jnp.where(lane < _E, g, neg)
    m1 = jnp.max(g, axis=1, keepdims=True)
    a1 = jnp.min(jnp.where(g == m1, lane, _EPAD), axis=1, keepdims=True)
    g2 = jnp.where(lane == a1, neg, g)
    m2 = jnp.max(g2, axis=1, keepdims=True)
    a2 = jnp.min(jnp.where(g2 == m2, lane, _EPAD), axis=1, keepdims=True)
    w1 = jax.nn.sigmoid(m1 - m2)
    w2 = 1.0 - w1
    oh1 = (lane == a1)
    oh2 = (lane == a2)
    cnt = jnp.sum(oh1.astype(jnp.float32) + oh2.astype(jnp.float32),
                  axis=0, keepdims=True)
    inv = 1.0 / jnp.maximum(cnt, 1.0)
    c = (w1 * oh1.astype(jnp.float32) + w2 * oh2.astype(jnp.float32)) * inv
    c_ref[...] = c
    le = jax.lax.broadcasted_iota(jnp.int32, (_T, _E * _EPAD), 1) // _EPAD
    mrep_ref[...] = ((le == a1) | (le == a2)).astype(jnp.float32)


def _main_kernel(x_ref, w1_ref, mrep_ref, s1_ref):
    w1 = w1_ref[0].astype(jnp.bfloat16)
    h = jax.lax.dot_general(x_ref[...], w1, (((1,), (1,)), ((), ())),
                            preferred_element_type=jnp.float32)
    h = h * jax.nn.sigmoid(h)
    masked = h * mrep_ref[:, 0:1]
    s1_ref[0, 0, :] = jnp.sum(masked, axis=0)


def _mean_kernel(s1_ref, w2_ref, m_ref):
    m_ref[0] = jax.lax.dot_general(
        s1_ref[0], w2_ref[0], (((1,), (1,)), ((), ())),
        preferred_element_type=jnp.float32)


def _combine_kernel(c_ref, m_ref, out_ref):
    cs = c_ref[:, 0:_E]
    out_ref[...] = jax.lax.dot_general(
        cs, m_ref[...], (((1,), (0,)), ((), ())),
        preferred_element_type=jnp.float32)


@jax.jit
def kernel(hidden_states, gate_w, W1, W2):
    b, s, d = hidden_states.shape
    x = hidden_states.reshape(-1, d)

    gw_pad = jnp.zeros((_EPAD, _D), jnp.float32).at[:_E].set(gate_w)
    c, mrep, x16 = pl.pallas_call(
        _routing_kernel,
        out_shape=(
            jax.ShapeDtypeStruct((_T, _EPAD), jnp.float32),
            jax.ShapeDtypeStruct((_T, _E * _EPAD), jnp.float32),
            jax.ShapeDtypeStruct((_T, _D), jnp.bfloat16),
        ),
    )(x, gw_pad)

    s1 = pl.pallas_call(
        _main_kernel,
        grid=(_E, _DFF // _DFF_TILE),
        in_specs=[
            pl.BlockSpec((_T, _D), lambda e, j: (0, 0)),
            pl.BlockSpec((1, _DFF_TILE, _D), lambda e, j: (e, j, 0)),
            pl.BlockSpec((_T, _EPAD), lambda e, j: (0, e)),
        ],
        out_specs=pl.BlockSpec((1, 1, _DFF_TILE), lambda e, j: (e, 0, j)),
        out_shape=jax.ShapeDtypeStruct((_E, 1, _DFF), jnp.float32),
    )(x16, W1, mrep)

    m = pl.pallas_call(
        _mean_kernel,
        grid=(_E,),
        in_specs=[
            pl.BlockSpec((1, 1, _DFF), lambda e: (e, 0, 0)),
            pl.BlockSpec((1, _D, _DFF), lambda e: (e, 0, 0)),
        ],
        out_specs=pl.BlockSpec((1, 1, _D), lambda e: (e, 0, 0)),
        out_shape=jax.ShapeDtypeStruct((_E, 1, _D), jnp.float32),
    )(s1, W2)
    m = m.reshape(_E, _D)

    out = pl.pallas_call(
        _combine_kernel,
        grid=(_T // _T_TILE,),
        in_specs=[
            pl.BlockSpec((_T_TILE, _EPAD), lambda t: (t, 0)),
            pl.BlockSpec((_E, _D), lambda t: (0, 0)),
        ],
        out_specs=pl.BlockSpec((_T_TILE, _D), lambda t: (t, 0)),
        out_shape=jax.ShapeDtypeStruct((_T, _D), jnp.float32),
    )(c, m)

    return out.reshape(b, s, d)


# speedup vs baseline: 2.1277x; 1.0985x over previous
import functools

import jax
import jax.numpy as jnp
from jax.experimental import pallas as pl

_T = 2048
_D = 768
_E = 8
_DFF = 3072
_EPAD = 128
_DFF_TILE = 512
_T_TILE = 512


def _routing_kernel(x_ref, gw_ref, c_ref, mrep_ref, x16_ref):
    x = x_ref[...]
    x16_ref[...] = x.astype(jnp.bfloat16)
    gw = gw_ref[...]
    g = jax.lax.dot_general(x, gw, (((1,), (1,)), ((), ())),
                            preferred_element_type=jnp.float32)
    lane = jax.lax.broadcasted_iota(jnp.int32, (_T, _EPAD), 1)
    neg = jnp.float32(-1e30)
    g = jnp.where(lane < _E, g, neg)
    m1 = jnp.max(g, axis=1, keepdims=True)
    a1 = jnp.min(jnp.where(g == m1, lane, _EPAD), axis=1, keepdims=True)
    g2 = jnp.where(lane == a1, neg, g)
    m2 = jnp.max(g2, axis=1, keepdims=True)
    a2 = jnp.min(jnp.where(g2 == m2, lane, _EPAD), axis=1, keepdims=True)
    w1 = jax.nn.sigmoid(m1 - m2)
    w2 = 1.0 - w1
    oh1 = (lane == a1)
    oh2 = (lane == a2)
    cnt = jnp.sum(oh1.astype(jnp.float32) + oh2.astype(jnp.float32),
                  axis=0, keepdims=True)
    inv = 1.0 / jnp.maximum(cnt, 1.0)
    c = (w1 * oh1.astype(jnp.float32) + w2 * oh2.astype(jnp.float32)) * inv
    c_ref[...] = c
    le = jax.lax.broadcasted_iota(jnp.int32, (_T, _E * _EPAD), 1) // _EPAD
    mrep_ref[...] = ((le == a1) | (le == a2)).astype(jnp.float32)


def _main_kernel(x_ref, w1_ref, w2_ref, mrep_ref, m_ref):
    j = pl.program_id(1)
    w1 = w1_ref[0].astype(jnp.bfloat16)
    h = jax.lax.dot_general(x_ref[...], w1, (((1,), (1,)), ((), ())),
                            preferred_element_type=jnp.float32)
    h = h * jax.nn.sigmoid(h)
    masked = h * mrep_ref[:, 0:1]
    s1 = jnp.sum(masked, axis=0)[None, :]
    pm = jax.lax.dot_general(s1, w2_ref[0], (((1,), (1,)), ((), ())),
                             preferred_element_type=jnp.float32)

    @pl.when(j == 0)
    def _():
        m_ref[0] = pm

    @pl.when(j > 0)
    def _():
        m_ref[0] += pm


def _combine_kernel(c_ref, m_ref, out_ref):
    cs = c_ref[:, 0:_E]
    out_ref[...] = jax.lax.dot_general(
        cs, m_ref[...], (((1,), (0,)), ((), ())),
        preferred_element_type=jnp.float32)


@jax.jit
def kernel(hidden_states, gate_w, W1, W2):
    b, s, d = hidden_states.shape
    x = hidden_states.reshape(-1, d)

    gw_pad = jnp.zeros((_EPAD, _D), jnp.float32).at[:_E].set(gate_w)
    c, mrep, x16 = pl.pallas_call(
        _routing_kernel,
        out_shape=(
            jax.ShapeDtypeStruct((_T, _EPAD), jnp.float32),
            jax.ShapeDtypeStruct((_T, _E * _EPAD), jnp.float32),
            jax.ShapeDtypeStruct((_T, _D), jnp.bfloat16),
        ),
    )(x, gw_pad)

    m = pl.pallas_call(
        _main_kernel,
        grid=(_E, _DFF // _DFF_TILE),
        in_specs=[
            pl.BlockSpec((_T, _D), lambda e, j: (0, 0)),
            pl.BlockSpec((1, _DFF_TILE, _D), lambda e, j: (e, j, 0)),
            pl.BlockSpec((1, _D, _DFF_TILE), lambda e, j: (e, 0, j)),
            pl.BlockSpec((_T, _EPAD), lambda e, j: (0, e)),
        ],
        out_specs=pl.BlockSpec((1, 1, _D), lambda e, j: (e, 0, 0)),
        out_shape=jax.ShapeDtypeStruct((_E, 1, _D), jnp.float32),
    )(x16, W1, W2, mrep)
    m = m.reshape(_E, _D)

    out = pl.pallas_call(
        _combine_kernel,
        grid=(_T // _T_TILE,),
        in_specs=[
            pl.BlockSpec((_T_TILE, _EPAD), lambda t: (t, 0)),
            pl.BlockSpec((_E, _D), lambda t: (0, 0)),
        ],
        out_specs=pl.BlockSpec((_T_TILE, _D), lambda t: (t, 0)),
        out_shape=jax.ShapeDtypeStruct((_T, _D), jnp.float32),
    )(c, m)

    return out.reshape(b, s, d)
